# baseline (speedup 1.0000x reference)
import functools

import jax
import jax.numpy as jnp
from jax import lax
from jax.experimental import pallas as pl
from jax.experimental.pallas import tpu as pltpu
from jax.experimental.pallas import tpu_sc as plsc

N = 10000
NP = 10240
E = 320000
EP = 327680
G = 128
SUB = 32
NSUB = G // SUB
NCHUNK = EP // G
JG = 2
CPT_G = NCHUNK // 32
GG = CPT_G // JG
JS = 1
CPT_S = NCHUNK // 32
GS = CPT_S // JS
IN_DIM = 128
HID = 64
N_ITERS = 10
N_GRAPHS = 16
OUT_DIM = 16

BE = 2048
BN = 1024

_mesh = plsc.VectorSubcoreMesh(core_axis_name="c", subcore_axis_name="s")


@functools.partial(
    pl.kernel,
    out_type=jax.ShapeDtypeStruct((2, EP, 128), jnp.float32),
    mesh=_mesh,
    scratch_types=[
        pltpu.VMEM((2, 2, JG, G), jnp.int32),
        pltpu.VMEM((JG * G, 128), jnp.float32),
        pltpu.VMEM((JG * G, 128), jnp.float32),
        pltpu.SemaphoreType.DMA,
        pltpu.SemaphoreType.DMA,
    ],
)
def _sc_gather(pq_hbm, ei_hbm, g_hbm, iv, rd, rs, sem_i, sem_g):
    c = lax.axis_index("c")
    s = lax.axis_index("s")
    row00 = (c * 16 + s) * CPT_G

    def idx_fetch(g, buf):
        pltpu.async_copy(ei_hbm.at[:, pl.ds(row00 + g * JG, JG)],
                         iv.at[buf], sem_i)

    def idx_wait(buf):
        pltpu.make_async_copy(
            ei_hbm.at[:, pl.ds(0, JG)], iv.at[buf], sem_i).wait()

    idx_fetch(0, 0)

    def body(g, carry):
        cur = lax.rem(g, 2)
        idx_wait(cur)
        idx_fetch(jnp.minimum(g + 1, GG - 1), 1 - cur)
        for j in range(JG):
            for k in range(NSUB):
                o = j * G + k * SUB
                pltpu.async_copy(
                    pq_hbm.at[iv.at[cur, 0, j, pl.ds(k * SUB, SUB)]],
                    rd.at[pl.ds(o, SUB)], sem_g)
                pltpu.async_copy(
                    pq_hbm.at[iv.at[cur, 1, j, pl.ds(k * SUB, SUB)]],
                    rs.at[pl.ds(o, SUB)], sem_g)
        for j in range(JG):
            for k in range(NSUB):
                o = j * G + k * SUB
                pltpu.make_async_copy(
                    pq_hbm.at[iv.at[cur, 0, j, pl.ds(k * SUB, SUB)]],
                    rd.at[pl.ds(o, SUB)], sem_g).wait()
                pltpu.make_async_copy(
                    pq_hbm.at[iv.at[cur, 1, j, pl.ds(k * SUB, SUB)]],
                    rs.at[pl.ds(o, SUB)], sem_g).wait()
        base = (row00 + g * JG) * G
        pltpu.sync_copy(rd, g_hbm.at[0, pl.ds(base, JG * G)])
        pltpu.sync_copy(rs, g_hbm.at[1, pl.ds(base, JG * G)])
        return carry

    lax.fori_loop(0, GG, body, 0)
    idx_wait(GG % 2)


@functools.partial(
    pl.kernel,
    out_type=jax.ShapeDtypeStruct((2, NP, 128), jnp.float32),
    mesh=_mesh,
    scratch_types=[
        pltpu.VMEM((2, JS, G), jnp.int32),
        pltpu.VMEM((2, JS * G, 128), jnp.float32),
        pltpu.VMEM_SHARED((NP, 128), jnp.float32),
        pltpu.SemaphoreType.DMA,
        pltpu.SemaphoreType.DMA,
    ],
)
def _sc_scatter(u_hbm, di_hbm, zeros_hbm, out_hbm, iv, uv, acc,
                sem_i, sem_s):
    c = lax.axis_index("c")
    s = lax.axis_index("s")
    rpt = NP // 16
    row00 = (c * 16 + s) * CPT_S

    def fetch(g, buf):
        row0 = row00 + g * JS
        pltpu.async_copy(di_hbm.at[pl.ds(row0, JS)], iv.at[buf], sem_i)
        pltpu.async_copy(u_hbm.at[pl.ds(row0 * G, JS * G)], uv.at[buf],
                         sem_i)

    def fetch_wait(buf):
        pltpu.make_async_copy(
            di_hbm.at[pl.ds(0, JS)], iv.at[buf], sem_i).wait()
        pltpu.make_async_copy(
            u_hbm.at[pl.ds(0, JS * G)], uv.at[buf], sem_i).wait()

    pltpu.sync_copy(zeros_hbm.at[pl.ds(s * rpt, rpt)],
                    acc.at[pl.ds(s * rpt, rpt)])
    fetch(0, 0)
    plsc.subcore_barrier()

    def body(g, carry):
        cur = lax.rem(g, 2)
        fetch_wait(cur)
        for j in range(JS):
            pltpu.async_copy(uv.at[cur, pl.ds(j * G, G)],
                             acc.at[iv.at[cur, j]], sem_s, add=True)
        fetch(jnp.minimum(g + 1, GS - 1), 1 - cur)
        for j in range(JS):
            pltpu.make_async_copy(uv.at[cur, pl.ds(j * G, G)],
                                  acc.at[iv.at[cur, j]], sem_s).wait()
        return carry

    lax.fori_loop(0, GS, body, 0)
    fetch_wait(GS % 2)
    plsc.subcore_barrier()
    pltpu.sync_copy(acc.at[pl.ds(s * rpt, rpt)],
                    out_hbm.at[c, pl.ds(s * rpt, rpt)])


def _sigmoid(v):
    return 1.0 / (1.0 + jnp.exp(-v))


def _tc_prep_body(x_ref, win_ref, bin_ref, ab_ref, xab_w_ref, b128_ref,
                  pq_ref, xab_ref):
    x = x_ref[...]
    xab = (jnp.dot(x, xab_w_ref[...], preferred_element_type=jnp.float32)
           + b128_ref[...])
    h0 = jnp.tanh(jnp.dot(x, win_ref[...], preferred_element_type=jnp.float32)
                  + bin_ref[...])
    xab_ref[...] = xab
    pq_ref[...] = (jnp.dot(h0, ab_ref[...],
                           preferred_element_type=jnp.float32) + xab)


def _tc_mid_body(g_ref, w2_ref, b2_ref, u_ref):
    g = g_ref[...]
    t = _sigmoid(g[0][:, :HID] + g[1][:, HID:])
    u = _sigmoid(jnp.dot(t, w2_ref[...], preferred_element_type=jnp.float32)
                 + b2_ref[...])
    u_ref[...] = jnp.concatenate(
        [u, jnp.zeros((u.shape[0], HID), jnp.float32)], axis=-1)


def _tc_pq_body(hp_ref, xab_ref, ab_ref, pq_ref):
    hn = (hp_ref[0] + hp_ref[1])[:, :HID]
    pq_ref[...] = (jnp.dot(hn, ab_ref[...],
                           preferred_element_type=jnp.float32)
                   + xab_ref[...])


def _tc_final_body(hp_ref, x_ref, oh_ref, woh_ref, wox_ref, bo_ref,
                   out_ref, acch, accx):
    i = pl.program_id(0)

    @pl.when(i == 0)
    def _():
        acch[...] = jnp.zeros_like(acch)
        accx[...] = jnp.zeros_like(accx)

    hn = hp_ref[0] + hp_ref[1]
    oh = oh_ref[...]
    dn = (((0,), (0,)), ((), ()))
    acch[...] += lax.dot_general(oh, hn, dn,
                                 preferred_element_type=jnp.float32)
    accx[...] += lax.dot_general(oh, x_ref[...], dn,
                                 preferred_element_type=jnp.float32)

    @pl.when(i == (NP // BN) - 1)
    def _():
        out_ref[...] = (
            jnp.dot(acch[...][:, :HID], woh_ref[...],
                    preferred_element_type=jnp.float32)
            + jnp.dot(accx[...], wox_ref[...],
                      preferred_element_type=jnp.float32)
            + bo_ref[...])


def _full(shape):
    return pl.BlockSpec(shape, lambda i: tuple(0 for _ in shape))


def kernel(x, edge_index, batch, W_in, b_in, W_c1, b_c1, W_c2, b_c2,
           W_out, b_out):
    f32 = jnp.float32
    xp = jnp.zeros((NP, IN_DIM), f32).at[:N].set(x)
    dst2d = jnp.concatenate(
        [edge_index[1], jnp.full((EP - E,), N, jnp.int32)]).reshape(NCHUNK, G)
    src2d = jnp.concatenate(
        [edge_index[0], jnp.zeros((EP - E,), jnp.int32)]).reshape(NCHUNK, G)
    ei = jnp.stack([dst2d, src2d])
    A1 = W_c1[:HID] - W_c1[192:256]
    A2 = W_c1[HID:192] - W_c1[256:]
    B1 = W_c1[192:256]
    B2 = W_c1[256:]
    ab = jnp.concatenate([A1, B1], axis=1)
    xab_w = jnp.concatenate([A2, B2], axis=1)
    b128 = jnp.concatenate(
        [b_c1, jnp.zeros((HID,), f32)]).reshape(1, 128)
    b2 = b_c2.reshape(1, HID)
    bo = b_out.reshape(1, OUT_DIM)
    woh = W_out[:HID]
    wox = W_out[HID:]
    batch_p = jnp.concatenate(
        [batch, jnp.full((NP - N,), N_GRAPHS, jnp.int32)])
    oh = (batch_p[:, None] == jnp.arange(N_GRAPHS)[None, :]).astype(f32)
    counts = jnp.maximum(oh.sum(axis=0), 1.0)
    ohs = oh / counts[None, :]
    zz = jnp.zeros((NP, 128), f32)

    nblk = NP // BN
    PQ, XAB = pl.pallas_call(
        _tc_prep_body,
        grid=(nblk,),
        in_specs=[
            pl.BlockSpec((BN, IN_DIM), lambda i: (i, 0)),
            _full((IN_DIM, HID)),
            _full((1, HID)),
            _full((HID, 128)),
            _full((IN_DIM, 128)),
            _full((1, 128)),
        ],
        out_specs=[pl.BlockSpec((BN, 128), lambda i: (i, 0))] * 2,
        out_shape=[jax.ShapeDtypeStruct((NP, 128), f32)] * 2,
    )(xp, W_in, b_in.reshape(1, HID), ab, xab_w, b128)

    mid = pl.pallas_call(
        _tc_mid_body,
        grid=(EP // BE,),
        in_specs=[
            pl.BlockSpec((2, BE, 128), lambda i: (0, i, 0)),
            _full((HID, HID)),
            _full((1, HID)),
        ],
        out_specs=pl.BlockSpec((BE, 128), lambda i: (i, 0)),
        out_shape=jax.ShapeDtypeStruct((EP, 128), f32),
    )

    pq_call = pl.pallas_call(
        _tc_pq_body,
        grid=(nblk,),
        in_specs=[
            pl.BlockSpec((2, BN, 128), lambda i: (0, i, 0)),
            pl.BlockSpec((BN, 128), lambda i: (i, 0)),
            _full((HID, 128)),
        ],
        out_specs=pl.BlockSpec((BN, 128), lambda i: (i, 0)),
        out_shape=jax.ShapeDtypeStruct((NP, 128), f32),
    )

    hp = None
    for it in range(N_ITERS):
        g = _sc_gather(PQ, ei)
        u = mid(g, W_c2, b2)
        hp = _sc_scatter(u, dst2d, zz)
        if it < N_ITERS - 1:
            PQ = pq_call(hp, XAB, ab)

    out = pl.pallas_call(
        _tc_final_body,
        grid=(nblk,),
        in_specs=[
            pl.BlockSpec((2, BN, 128), lambda i: (0, i, 0)),
            pl.BlockSpec((BN, IN_DIM), lambda i: (i, 0)),
            pl.BlockSpec((BN, N_GRAPHS), lambda i: (i, 0)),
            _full((HID, OUT_DIM)),
            _full((IN_DIM, OUT_DIM)),
            _full((1, OUT_DIM)),
        ],
        out_specs=_full((N_GRAPHS, OUT_DIM)),
        out_shape=jax.ShapeDtypeStruct((N_GRAPHS, OUT_DIM), f32),
        scratch_shapes=[
            pltpu.VMEM((N_GRAPHS, 128), f32),
            pltpu.VMEM((N_GRAPHS, IN_DIM), f32),
        ],
    )(hp, xp, ohs, woh, wox, bo)
    return out


# speedup vs baseline: 1.1122x; 1.1122x over previous
import functools

import jax
import jax.numpy as jnp
from jax import lax
from jax.experimental import pallas as pl
from jax.experimental.pallas import tpu as pltpu
from jax.experimental.pallas import tpu_sc as plsc

N = 10000
NP = 10240
E = 320000
EP = 327680
G = 128
NCHUNK = EP // G
JG = 8
CPT_G = NCHUNK // 16
GG = CPT_G // JG
JS = 1
CPT_S = NCHUNK // 32
GS = CPT_S // JS
IN_DIM = 128
HID = 64
N_ITERS = 10
N_GRAPHS = 16
OUT_DIM = 16

BE = 2048
BN = 1024

_mesh = plsc.VectorSubcoreMesh(core_axis_name="c", subcore_axis_name="s")


@functools.partial(
    pl.kernel,
    out_type=jax.ShapeDtypeStruct((2, EP, HID), jnp.bfloat16),
    mesh=_mesh,
    compiler_params=pltpu.CompilerParams(use_tc_tiling_on_sc=False),
    scratch_types=[
        pltpu.VMEM((2, JG, G), jnp.int32),
        pltpu.VMEM((JG * G, HID), jnp.bfloat16),
        pltpu.SemaphoreType.DMA,
        pltpu.SemaphoreType.DMA,
    ],
)
def _sc_gather(t_hbm, ei_hbm, gg_hbm, iv, rows, sem_i, sem_g):
    c = lax.axis_index("c")
    s = lax.axis_index("s")
    row00 = s * CPT_G

    def idx_fetch(g, buf):
        pltpu.async_copy(ei_hbm.at[c, pl.ds(row00 + g * JG, JG)],
                         iv.at[buf], sem_i)

    def idx_wait(buf):
        pltpu.make_async_copy(
            ei_hbm.at[0, pl.ds(0, JG)], iv.at[buf], sem_i).wait()

    idx_fetch(0, 0)

    def body(g, carry):
        cur = lax.rem(g, 2)
        idx_wait(cur)
        idx_fetch(jnp.minimum(g + 1, GG - 1), 1 - cur)
        for j in range(JG):
            pltpu.async_copy(t_hbm.at[iv.at[cur, j]],
                             rows.at[pl.ds(j * G, G)], sem_g)
        for j in range(JG):
            pltpu.make_async_copy(t_hbm.at[iv.at[cur, j]],
                                  rows.at[pl.ds(j * G, G)], sem_g).wait()
        base = (row00 + g * JG) * G
        pltpu.sync_copy(rows, gg_hbm.at[c, pl.ds(base, JG * G)])
        return carry

    lax.fori_loop(0, GG, body, 0)
    idx_wait(GG % 2)


@functools.partial(
    pl.kernel,
    out_type=jax.ShapeDtypeStruct((2, NP, 128), jnp.float32),
    mesh=_mesh,
    scratch_types=[
        pltpu.VMEM((2, JS, G), jnp.int32),
        pltpu.VMEM((2, JS * G, 128), jnp.float32),
        pltpu.VMEM_SHARED((NP, 128), jnp.float32),
        pltpu.SemaphoreType.DMA,
        pltpu.SemaphoreType.DMA,
    ],
)
def _sc_scatter(u_hbm, di_hbm, zeros_hbm, out_hbm, iv, uv, acc,
                sem_i, sem_s):
    c = lax.axis_index("c")
    s = lax.axis_index("s")
    rpt = NP // 16
    row00 = (c * 16 + s) * CPT_S

    def fetch(g, buf):
        row0 = row00 + g * JS
        pltpu.async_copy(di_hbm.at[pl.ds(row0, JS)], iv.at[buf], sem_i)
        pltpu.async_copy(u_hbm.at[pl.ds(row0 * G, JS * G)], uv.at[buf],
                         sem_i)

    def fetch_wait(buf):
        pltpu.make_async_copy(
            di_hbm.at[pl.ds(0, JS)], iv.at[buf], sem_i).wait()
        pltpu.make_async_copy(
            u_hbm.at[pl.ds(0, JS * G)], uv.at[buf], sem_i).wait()

    pltpu.sync_copy(zeros_hbm.at[pl.ds(s * rpt, rpt)],
                    acc.at[pl.ds(s * rpt, rpt)])
    fetch(0, 0)
    plsc.subcore_barrier()

    def body(g, carry):
        cur = lax.rem(g, 2)
        fetch_wait(cur)
        for j in range(JS):
            pltpu.async_copy(uv.at[cur, pl.ds(j * G, G)],
                             acc.at[iv.at[cur, j]], sem_s, add=True)
        fetch(jnp.minimum(g + 1, GS - 1), 1 - cur)
        for j in range(JS):
            pltpu.make_async_copy(uv.at[cur, pl.ds(j * G, G)],
                                  acc.at[iv.at[cur, j]], sem_s).wait()
        return carry

    lax.fori_loop(0, GS, body, 0)
    fetch_wait(GS % 2)
    plsc.subcore_barrier()
    pltpu.sync_copy(acc.at[pl.ds(s * rpt, rpt)],
                    out_hbm.at[c, pl.ds(s * rpt, rpt)])


def _sigmoid(v):
    return 1.0 / (1.0 + jnp.exp(-v))


def _tc_prep_body(x_ref, win_ref, bin_ref, ab_ref, xab_w_ref, b128_ref,
                  pq_ref, xab_ref):
    x = x_ref[...]
    xab = (jnp.dot(x, xab_w_ref[...], preferred_element_type=jnp.float32)
           + b128_ref[...])
    h0 = jnp.tanh(jnp.dot(x, win_ref[...], preferred_element_type=jnp.float32)
                  + bin_ref[...])
    xab_ref[...] = xab
    pq = (jnp.dot(h0, ab_ref[...], preferred_element_type=jnp.float32)
          + xab).astype(jnp.bfloat16)
    pq_ref[0] = pq[:, :HID]
    pq_ref[1] = pq[:, HID:]


def _tc_mid_body(g_ref, w2_ref, b2_ref, u_ref):
    g = g_ref[...]
    t = _sigmoid(g[0].astype(jnp.float32) + g[1].astype(jnp.float32))
    u = _sigmoid(jnp.dot(t.astype(jnp.bfloat16), w2_ref[...],
                         preferred_element_type=jnp.float32)
                 + b2_ref[...])
    u_ref[...] = jnp.concatenate(
        [u, jnp.zeros((u.shape[0], HID), jnp.float32)], axis=-1)


def _tc_pq_body(hp_ref, xab_ref, ab_ref, pq_ref):
    hn = (hp_ref[0] + hp_ref[1])[:, :HID]
    pq = (jnp.dot(hn, ab_ref[...], preferred_element_type=jnp.float32)
          + xab_ref[...]).astype(jnp.bfloat16)
    pq_ref[0] = pq[:, :HID]
    pq_ref[1] = pq[:, HID:]


def _tc_final_body(hp_ref, x_ref, oh_ref, woh_ref, wox_ref, bo_ref,
                   out_ref, acch, accx):
    i = pl.program_id(0)

    @pl.when(i == 0)
    def _():
        acch[...] = jnp.zeros_like(acch)
        accx[...] = jnp.zeros_like(accx)

    hn = hp_ref[0] + hp_ref[1]
    oh = oh_ref[...]
    dn = (((0,), (0,)), ((), ()))
    acch[...] += lax.dot_general(oh, hn, dn,
                                 preferred_element_type=jnp.float32)
    accx[...] += lax.dot_general(oh, x_ref[...], dn,
                                 preferred_element_type=jnp.float32)

    @pl.when(i == (NP // BN) - 1)
    def _():
        out_ref[...] = (
            jnp.dot(acch[...][:, :HID], woh_ref[...],
                    preferred_element_type=jnp.float32)
            + jnp.dot(accx[...], wox_ref[...],
                      preferred_element_type=jnp.float32)
            + bo_ref[...])


def _full(shape):
    return pl.BlockSpec(shape, lambda i: tuple(0 for _ in shape))


def kernel(x, edge_index, batch, W_in, b_in, W_c1, b_c1, W_c2, b_c2,
           W_out, b_out):
    f32 = jnp.float32
    xp = jnp.zeros((NP, IN_DIM), f32).at[:N].set(x)
    dst2d = jnp.concatenate(
        [edge_index[1], jnp.full((EP - E,), N, jnp.int32)]).reshape(NCHUNK, G)
    src2d = jnp.concatenate(
        [edge_index[0], jnp.zeros((EP - E,), jnp.int32)]).reshape(NCHUNK, G)
    ei = jnp.stack([dst2d, src2d + NP])
    A1 = W_c1[:HID] - W_c1[192:256]
    A2 = W_c1[HID:192] - W_c1[256:]
    B1 = W_c1[192:256]
    B2 = W_c1[256:]
    ab = jnp.concatenate([A1, B1], axis=1)
    xab_w = jnp.concatenate([A2, B2], axis=1)
    b128 = jnp.concatenate(
        [b_c1, jnp.zeros((HID,), f32)]).reshape(1, 128)
    w2b = W_c2.astype(jnp.bfloat16)
    b2 = b_c2.reshape(1, HID)
    bo = b_out.reshape(1, OUT_DIM)
    woh = W_out[:HID]
    wox = W_out[HID:]
    batch_p = jnp.concatenate(
        [batch, jnp.full((NP - N,), N_GRAPHS, jnp.int32)])
    oh = (batch_p[:, None] == jnp.arange(N_GRAPHS)[None, :]).astype(f32)
    counts = jnp.maximum(oh.sum(axis=0), 1.0)
    ohs = oh / counts[None, :]
    zz = jnp.zeros((NP, 128), f32)

    nblk = NP // BN
    PQ, XAB = pl.pallas_call(
        _tc_prep_body,
        grid=(nblk,),
        in_specs=[
            pl.BlockSpec((BN, IN_DIM), lambda i: (i, 0)),
            _full((IN_DIM, HID)),
            _full((1, HID)),
            _full((HID, 128)),
            _full((IN_DIM, 128)),
            _full((1, 128)),
        ],
        out_specs=[pl.BlockSpec((2, BN, HID), lambda i: (0, i, 0)),
                   pl.BlockSpec((BN, 128), lambda i: (i, 0))],
        out_shape=[jax.ShapeDtypeStruct((2, NP, HID), jnp.bfloat16),
                   jax.ShapeDtypeStruct((NP, 128), f32)],
    )(xp, W_in, b_in.reshape(1, HID), ab, xab_w, b128)

    mid = pl.pallas_call(
        _tc_mid_body,
        grid=(EP // BE,),
        in_specs=[
            pl.BlockSpec((2, BE, HID), lambda i: (0, i, 0)),
            _full((HID, HID)),
            _full((1, HID)),
        ],
        out_specs=pl.BlockSpec((BE, 128), lambda i: (i, 0)),
        out_shape=jax.ShapeDtypeStruct((EP, 128), f32),
    )

    pq_call = pl.pallas_call(
        _tc_pq_body,
        grid=(nblk,),
        in_specs=[
            pl.BlockSpec((2, BN, 128), lambda i: (0, i, 0)),
            pl.BlockSpec((BN, 128), lambda i: (i, 0)),
            _full((HID, 128)),
        ],
        out_specs=pl.BlockSpec((2, BN, HID), lambda i: (0, i, 0)),
        out_shape=jax.ShapeDtypeStruct((2, NP, HID), jnp.bfloat16),
    )

    hp = None
    for it in range(N_ITERS):
        g = _sc_gather(PQ.reshape(2 * NP, HID), ei)
        u = mid(g, w2b, b2)
        hp = _sc_scatter(u, dst2d, zz)
        if it < N_ITERS - 1:
            PQ = pq_call(hp, XAB, ab)

    out = pl.pallas_call(
        _tc_final_body,
        grid=(nblk,),
        in_specs=[
            pl.BlockSpec((2, BN, 128), lambda i: (0, i, 0)),
            pl.BlockSpec((BN, IN_DIM), lambda i: (i, 0)),
            pl.BlockSpec((BN, N_GRAPHS), lambda i: (i, 0)),
            _full((HID, OUT_DIM)),
            _full((IN_DIM, OUT_DIM)),
            _full((1, OUT_DIM)),
        ],
        out_specs=_full((N_GRAPHS, OUT_DIM)),
        out_shape=jax.ShapeDtypeStruct((N_GRAPHS, OUT_DIM), f32),
        scratch_shapes=[
            pltpu.VMEM((N_GRAPHS, 128), f32),
            pltpu.VMEM((N_GRAPHS, IN_DIM), f32),
        ],
    )(hp, xp, ohs, woh, wox, bo)
    return out
